# Initial kernel scaffold; baseline (speedup 1.0000x reference)
import jax
import jax.numpy as jnp
from jax.experimental import pallas as pl


def kernel(head, tail, edge_index, edge_type, ent_emb, rel_embs, lin_W, lin_b, rel_lin_W, rel_lin_b, attn_W, attn_b, Wr_W, Wr_b):
    raise NotImplementedError("write your pallas kernel here")


# speedup vs baseline: 2.8032x; 2.8032x over previous
import functools

import jax
import jax.numpy as jnp
from jax import lax
from jax.experimental import pallas as pl
from jax.experimental.pallas import tpu as pltpu
from jax.experimental.pallas import tpu_sc as plsc

N_ENT = 10000
N_PAD = 10240
N_DIM = 64
N_LAYER = 2
N_RELP = 32
BN = 2
BD = BN * N_DIM
E_EDGES = 160000
CH = 128
N_CHUNKS = E_EDGES // CH
N_WORKERS = 32
ROWS_PER_TILE = N_PAD // 16
ROW_BLK = 2048


def _rspmm(src, typ, dst, h, rel, zeros):
  mesh = plsc.VectorSubcoreMesh(core_axis_name="c", subcore_axis_name="s",
                                num_cores=2, num_subcores=16)

  @functools.partial(
      pl.kernel,
      out_type=jax.ShapeDtypeStruct((2 * N_PAD, BD), jnp.float32),
      mesh=mesh,
      scratch_types=[
          pltpu.VMEM((CH,), jnp.int32),
          pltpu.VMEM((CH,), jnp.int32),
          pltpu.VMEM((CH,), jnp.int32),
          pltpu.VMEM((CH, BD), jnp.float32),
          pltpu.VMEM((CH, BD), jnp.float32),
          pltpu.VMEM_SHARED((N_PAD, BD), jnp.float32),
          pltpu.SemaphoreType.DMA,
          pltpu.SemaphoreType.DMA,
      ],
  )
  def body(src_hbm, typ_hbm, dst_hbm, h_hbm, rel_hbm, z_hbm, out_hbm,
           sv, tv, dv, hv, rv, agg, sem_a, sem_b):
    cid = lax.axis_index("c")
    sid = lax.axis_index("s")
    wid = sid * 2 + cid

    r0 = sid * ROWS_PER_TILE
    pltpu.sync_copy(z_hbm.at[pl.ds(r0, ROWS_PER_TILE)],
                    agg.at[pl.ds(r0, ROWS_PER_TILE)])
    plsc.subcore_barrier()

    n_my = 39 + jnp.where(wid < N_CHUNKS - 39 * N_WORKERS, 1, 0)

    def chunk_body(k, carry):
      off = (k * N_WORKERS + wid) * CH
      pltpu.sync_copy(src_hbm.at[pl.ds(off, CH)], sv)
      pltpu.sync_copy(typ_hbm.at[pl.ds(off, CH)], tv)
      pltpu.sync_copy(dst_hbm.at[pl.ds(off, CH)], dv)
      ca = pltpu.async_copy(h_hbm.at[sv], hv, sem_a)
      cb = pltpu.async_copy(rel_hbm.at[tv], rv, sem_b)
      ca.wait()
      cb.wait()

      def mul_row(j, c2):
        for c in range(BD // 16):
          hv[j, pl.ds(c * 16, 16)] = (hv[j, pl.ds(c * 16, 16)] *
                                      rv[j, pl.ds(c * 16, 16)])
        return c2

      lax.fori_loop(0, CH, mul_row, 0, unroll=False)
      pltpu.sync_copy(hv, agg.at[dv], add=True)
      return carry

    lax.fori_loop(0, n_my, chunk_body, 0, unroll=False)
    plsc.subcore_barrier()

    pltpu.sync_copy(agg.at[pl.ds(r0, ROWS_PER_TILE)],
                    out_hbm.at[pl.ds(cid * N_PAD + r0, ROWS_PER_TILE)])

  out = body(src, typ, dst, h, rel, zeros)
  return out.reshape(2, N_PAD, BD)


def _relw_body(htT_ref, w1T_ref, b1T_ref, w2T_ref, b2T_ref, emb_ref, o_ref):
  xT = lax.dot_general(w1T_ref[0], htT_ref[...], (((1,), (0,)), ((), ())),
                       preferred_element_type=jnp.float32)
  xT = jnp.maximum(xT + b1T_ref[0], 0.0)
  wT = lax.dot_general(w2T_ref[0], xT, (((1,), (0,)), ((), ())),
                       preferred_element_type=jnp.float32)
  wT = jax.nn.sigmoid(wT + b2T_ref[0])
  e = emb_ref[0]
  o_ref[0] = jnp.concatenate([wT[:, 0:1] * e, wT[:, 1:2] * e], axis=1)


def _rel_tables(htT, w1T, b1T, w2T, b2T, embp):
  return pl.pallas_call(
      _relw_body,
      grid=(N_LAYER,),
      in_specs=[
          pl.BlockSpec((BD, 8), lambda l: (0, 0)),
          pl.BlockSpec((1, 8, BD), lambda l: (l, 0, 0)),
          pl.BlockSpec((1, 8, 1), lambda l: (l, 0, 0)),
          pl.BlockSpec((1, N_RELP, 8), lambda l: (l, 0, 0)),
          pl.BlockSpec((1, N_RELP, 1), lambda l: (l, 0, 0)),
          pl.BlockSpec((1, N_RELP, N_DIM), lambda l: (l, 0, 0)),
      ],
      out_specs=pl.BlockSpec((1, N_RELP, BD), lambda l: (l, 0, 0)),
      out_shape=jax.ShapeDtypeStruct((N_LAYER, N_RELP, BD), jnp.float32),
  )(htT, w1T, b1T, w2T, b2T, embp)


def _init_body(idx_ref, emb_ref, o_ref):
  i = pl.program_id(0)
  rows = jax.lax.broadcasted_iota(jnp.int32, (ROW_BLK, 1), 0) + i * ROW_BLK
  e0 = emb_ref[0:1, :]
  e1 = emb_ref[1:2, :]
  left = jnp.where(rows == idx_ref[0], e0, 0.0)
  right = jnp.where(rows == idx_ref[1], e1, 0.0)
  o_ref[...] = jnp.concatenate([left, right], axis=1)


def _init_hidden(idx, emb):
  return pl.pallas_call(
      _init_body,
      grid=(N_PAD // ROW_BLK,),
      in_specs=[
          pl.BlockSpec(memory_space=pltpu.SMEM),
          pl.BlockSpec((BN, N_DIM), lambda i: (0, 0)),
      ],
      out_specs=pl.BlockSpec((ROW_BLK, BD), lambda i: (i, 0)),
      out_shape=jax.ShapeDtypeStruct((N_PAD, BD), jnp.float32),
  )(idx, emb)


def _lin_body(p_ref, w_ref, b_ref, o_ref):
  a = p_ref[0] + p_ref[1]
  w = w_ref[...]
  b = b_ref[...]
  x1 = lax.dot_general(a[:, :N_DIM], w, (((1,), (0,)), ((), ())),
                       preferred_element_type=jnp.float32)
  x2 = lax.dot_general(a[:, N_DIM:], w, (((1,), (0,)), ((), ())),
                       preferred_element_type=jnp.float32)
  o_ref[...] = jnp.concatenate(
      [jnp.maximum(x1 + b, 0.0), jnp.maximum(x2 + b, 0.0)], axis=1)


def _combine_lin(parts, w, b):
  return pl.pallas_call(
      _lin_body,
      grid=(N_PAD // ROW_BLK,),
      in_specs=[
          pl.BlockSpec((2, ROW_BLK, BD), lambda i: (0, i, 0)),
          pl.BlockSpec((N_DIM, N_DIM), lambda i: (0, 0)),
          pl.BlockSpec((1, N_DIM), lambda i: (0, 0)),
      ],
      out_specs=pl.BlockSpec((ROW_BLK, BD), lambda i: (i, 0)),
      out_shape=jax.ShapeDtypeStruct((N_PAD, BD), jnp.float32),
  )(parts, w, b)


def _score_body(e_ref, w_ref, b_ref, o_ref):
  o_ref[...] = lax.dot_general(e_ref[...], w_ref[...],
                               (((1,), (0,)), ((), ())),
                               preferred_element_type=jnp.float32) + b_ref[...]


def _scores(embp, wp, bp):
  return pl.pallas_call(
      _score_body,
      out_shape=jax.ShapeDtypeStruct((8, 128), jnp.float32),
  )(embp, wp, bp)


@jax.jit
def _run(head, tail, edge_index, edge_type, ent_emb, rel_embs, lin_W, lin_b,
         rel_lin_W, rel_lin_b, attn_W, attn_b, Wr_W, Wr_b):
  dst = edge_index[0].astype(jnp.int32)
  src = edge_index[1].astype(jnp.int32)
  typ = edge_type.astype(jnp.int32)

  head_embed = jnp.take(ent_emb, head, axis=0)
  tail_embed = jnp.take(ent_emb, tail, axis=0)
  ht = jnp.concatenate([head_embed, tail_embed], axis=-1)

  htT = jnp.transpose(ht).reshape(BD, 2)
  htT = jnp.pad(htT, ((0, 0), (0, 6)))
  w1T = jnp.pad(jnp.transpose(rel_lin_W, (0, 2, 1)), ((0, 0), (0, 3), (0, 0)))
  b1T = jnp.pad(rel_lin_b, ((0, 0), (0, 3)))[:, :, None]
  w2T = jnp.pad(jnp.transpose(attn_W, (0, 2, 1)),
                ((0, 0), (0, N_RELP - attn_W.shape[2]), (0, 3)))
  b2T = jnp.pad(attn_b, ((0, 0), (0, N_RELP - attn_b.shape[1])))[:, :, None]
  embp = jnp.pad(rel_embs, ((0, 0), (0, N_RELP - rel_embs.shape[1]), (0, 0)))
  rel_tab = _rel_tables(htT, w1T, b1T, w2T, b2T, embp)

  zeros = jnp.zeros((N_PAD, BD), jnp.float32)
  lin_bb = lin_b.reshape(N_LAYER, 1, N_DIM)

  def propagate(init_idx, init_emb):
    h = _init_hidden(init_idx.astype(jnp.int32), init_emb)
    for l in range(N_LAYER):
      parts = _rspmm(src, typ, dst, h, rel_tab[l], zeros)
      h = _combine_lin(parts, lin_W[l], lin_bb[l])
    return h

  ar = jnp.arange(BN)
  ht_t = _run_pick(propagate(head, head_embed), tail)
  hh_t = _run_pick(propagate(tail, tail_embed), head)

  emb_cat = jnp.concatenate([head_embed, tail_embed, hh_t, ht_t], axis=1)
  embp8 = jnp.pad(emb_cat, ((0, 6), (0, 0)))
  wp = jnp.pad(Wr_W, ((0, 0), (0, 128 - Wr_W.shape[1])))
  bp = jnp.pad(Wr_b, (0, 128 - Wr_b.shape[0])).reshape(1, 128)
  sc = _scores(embp8, wp, bp)
  return sc[:BN, :Wr_W.shape[1]]


def _run_pick(hid, idx):
  rows = jnp.take(hid, idx, axis=0)
  return jnp.stack([rows[0, :N_DIM], rows[1, N_DIM:]], axis=0)


def kernel(head, tail, edge_index, edge_type, ent_emb, rel_embs, lin_W, lin_b,
           rel_lin_W, rel_lin_b, attn_W, attn_b, Wr_W, Wr_b):
  return _run(head, tail, edge_index, edge_type, ent_emb, rel_embs, lin_W,
              lin_b, rel_lin_W, rel_lin_b, attn_W, attn_b, Wr_W, Wr_b)
